# baseline (speedup 1.0000x reference)
import functools

import jax
import jax.numpy as jnp
from jax import lax
from jax.experimental import pallas as pl
from jax.experimental.pallas import tpu as pltpu
from jax.experimental.pallas import tpu_sc as plsc

N_NODES = 10000
N_EDGES = 320000
INP_DIM = 128
HID_DIM = 256
OUT_DIM = 128
APPNP_K = 10
APPNP_ALPHA = 0.1

NC = 2
NS = 16
NW = NC * NS
D = 128
NP = 10240
NTB = NP // NW
ECH = 128
E_PAD = 327680
EPT = E_PAD // NS
PCH_B = 1024
PCH_F = 4096
NCH_F = E_PAD // PCH_F
BCH = 11520
NCH = BCH // ECH
B_TOT = NW * BCH

_mesh = plsc.VectorSubcoreMesh(core_axis_name="c", subcore_axis_name="s")
_sc_params = pltpu.CompilerParams(needs_layout_passes=False)


def _iota16():
    return lax.iota(jnp.int32, 16)


def _rsqrt16(d):
    y = jnp.full((16,), 0.1, jnp.float32)
    for _ in range(12):
        y = y * (1.5 - 0.5 * d * y * y)
    return y


_MLP_BLK = 1024


def _mlp_body(x_ref, w1_ref, b1_ref, w2_ref, b2_ref, out_ref):
    xb = x_ref[...]
    h1 = lax.dot_general(xb, w1_ref[...], (((1,), (1,)), ((), ())),
                         preferred_element_type=jnp.float32) + b1_ref[...]
    h1 = jnp.where(h1 > 0, h1, jnp.exp(jnp.minimum(h1, 0.0)) - 1.0)
    out_ref[...] = lax.dot_general(h1, w2_ref[...], (((1,), (1,)), ((), ())),
                                   preferred_element_type=jnp.float32) + b2_ref[...]


def _mlp(x_pad, W1, b1, W2, b2):
    return pl.pallas_call(
        _mlp_body,
        grid=(NP // _MLP_BLK,),
        in_specs=[
            pl.BlockSpec((_MLP_BLK, INP_DIM), lambda i: (i, 0)),
            pl.BlockSpec((HID_DIM, INP_DIM), lambda i: (0, 0)),
            pl.BlockSpec((1, HID_DIM), lambda i: (0, 0)),
            pl.BlockSpec((OUT_DIM, HID_DIM), lambda i: (0, 0)),
            pl.BlockSpec((1, OUT_DIM), lambda i: (0, 0)),
        ],
        out_specs=pl.BlockSpec((_MLP_BLK, D), lambda i: (i, 0)),
        out_shape=jax.ShapeDtypeStruct((NP, D), jnp.float32),
    )(x_pad, W1, b1.reshape(1, HID_DIM), W2, b2.reshape(1, OUT_DIM))


def _prep_body(src_hbm, dst_hbm, w_hbm, dst2_hbm, w2_hbm,
               a_hbm, sb_hbm, db_hbm, cb_hbm,
               deg_sp, a_sp, d2_v, w2_v, zb_v, av_v,
               sf0_v, sf1_v, df0_v, df1_v, wf0_v, wf1_v,
               ss_v, ds_v, cs_v, semb, semf0, semf1):
    sid = lax.axis_index("s")
    cid = lax.axis_index("c")
    base_n = sid * (NP // NS)
    wid = cid * NS + sid
    sfv = (sf0_v, sf1_v)
    dfv = (df0_v, df1_v)
    wfv = (wf0_v, wf1_v)
    semf = (semf0, semf1)

    for g in range((NP // NS) // 16):
        zb_v[pl.ds(g * 16, 16)] = jnp.zeros((16,), jnp.float32)
    pltpu.sync_copy(zb_v, deg_sp.at[pl.ds(base_n, NP // NS)])
    plsc.subcore_barrier()

    @pl.loop(0, EPT // PCH_B)
    def _deg_chunk(k):
        off = sid * EPT + k * PCH_B
        row = pl.multiple_of(off // ECH, 8)
        pltpu.sync_copy(dst2_hbm.at[pl.ds(row, 8)], d2_v)
        pltpu.sync_copy(w2_hbm.at[pl.ds(row, 8)], w2_v)
        for j in range(8):
            pltpu.async_copy(w2_v.at[j], deg_sp.at[d2_v.at[j]], semb,
                             add=True)
        for j in range(8):
            pltpu.make_async_copy(w2_v.at[j], deg_sp.at[pl.ds(0, ECH)],
                                  semb).wait()

    plsc.subcore_barrier()

    pltpu.sync_copy(deg_sp.at[pl.ds(base_n, NP // NS)], zb_v)
    for g in range((NP // NS) // 16):
        d16 = zb_v[pl.ds(g * 16, 16)] + 1.0
        zb_v[pl.ds(g * 16, 16)] = _rsqrt16(d16)
    pltpu.sync_copy(zb_v, a_sp.at[pl.ds(base_n, NP // NS)])
    pltpu.sync_copy(zb_v, a_hbm.at[pl.ds(base_n, NP // NS)])
    plsc.subcore_barrier()

    pltpu.sync_copy(a_sp, av_v)

    @pl.loop(0, BCH // 16)
    def _zstage(g):
        z16i = jnp.zeros((16,), jnp.int32)
        ss_v[pl.ds(g * 16, 16)] = z16i
        ds_v[pl.ds(g * 16, 16)] = z16i
        cs_v[pl.ds(g * 16, 16)] = jnp.zeros((16,), jnp.float32)

    def _issue_fstage(ch, p):
        pltpu.async_copy(src_hbm.at[pl.ds(ch * PCH_F, PCH_F)], sfv[p],
                         semf[p])
        pltpu.async_copy(dst_hbm.at[pl.ds(ch * PCH_F, PCH_F)], dfv[p],
                         semf[p])
        pltpu.async_copy(w_hbm.at[pl.ds(ch * PCH_F, PCH_F)], wfv[p],
                         semf[p])

    def _wait_fstage(p):
        pltpu.make_async_copy(src_hbm.at[pl.ds(0, PCH_F)], sfv[p],
                              semf[p]).wait()
        pltpu.make_async_copy(dst_hbm.at[pl.ds(0, PCH_F)], dfv[p],
                              semf[p]).wait()
        pltpu.make_async_copy(w_hbm.at[pl.ds(0, PCH_F)], wfv[p],
                              semf[p]).wait()

    _issue_fstage(0, 0)
    myb = jnp.full((16,), wid, jnp.int32)
    mybase = myb * NTB

    def _fchunk(ch, p, cnt):
        _wait_fstage(p)

        @pl.when(ch + 1 < NCH_F)
        def _pre():
            _issue_fstage(ch + 1, 1 - p)

        @pl.loop(0, PCH_F // 16, init_carry=cnt)
        def _grp(g, cnt):
            s16 = sfv[p][pl.ds(g * 16, 16)]
            d16 = dfv[p][pl.ds(g * 16, 16)]
            w16 = wfv[p][pl.ds(g * 16, 16)]
            b16 = ((d16 >> 6) * 52429) >> 18
            mask = b16 == myb
            mi = mask.astype(jnp.int32)
            pos = cnt + (plsc.cumsum(mi) - mi)
            mask = jnp.logical_and(mask, pos < BCH)
            dl16 = d16 - mybase
            asrc = plsc.load_gather(av_v, [s16])
            adst = plsc.load_gather(av_v, [d16])
            c16 = asrc * adst * w16
            plsc.store_scatter(ss_v, [pos], s16, mask=mask)
            plsc.store_scatter(ds_v, [pos], dl16, mask=mask)
            plsc.store_scatter(cs_v, [pos], c16, mask=mask)
            return cnt + plsc.all_reduce_population_count(mask)

        return _grp

    cnt = jnp.zeros((16,), jnp.int32)

    @pl.loop(0, NCH_F // 2, init_carry=cnt)
    def _floop(t, cnt):
        cnt = _fchunk(t * 2, 0, cnt)
        cnt = _fchunk(t * 2 + 1, 1, cnt)
        return cnt

    pltpu.sync_copy(ss_v, sb_hbm.at[pl.ds(wid * BCH, BCH)])
    pltpu.sync_copy(ds_v, db_hbm.at[pl.ds(wid * BCH, BCH)])
    pltpu.sync_copy(cs_v, cb_hbm.at[pl.ds(wid * BCH, BCH)])


_prep = pl.kernel(
    _prep_body,
    out_type=(
        jax.ShapeDtypeStruct((NP,), jnp.float32),
        jax.ShapeDtypeStruct((B_TOT,), jnp.int32),
        jax.ShapeDtypeStruct((B_TOT,), jnp.int32),
        jax.ShapeDtypeStruct((B_TOT,), jnp.float32),
    ),
    mesh=_mesh,
    compiler_params=_sc_params,
    scratch_types=[
        pltpu.VMEM_SHARED((NP,), jnp.float32),
        pltpu.VMEM_SHARED((NP,), jnp.float32),
        pltpu.VMEM((8, ECH), jnp.int32),
        pltpu.VMEM((8, ECH), jnp.float32),
        pltpu.VMEM((NP // NS,), jnp.float32),
        pltpu.VMEM((NP,), jnp.float32),
        pltpu.VMEM((PCH_F,), jnp.int32),
        pltpu.VMEM((PCH_F,), jnp.int32),
        pltpu.VMEM((PCH_F,), jnp.int32),
        pltpu.VMEM((PCH_F,), jnp.int32),
        pltpu.VMEM((PCH_F,), jnp.float32),
        pltpu.VMEM((PCH_F,), jnp.float32),
        pltpu.VMEM((BCH,), jnp.int32),
        pltpu.VMEM((BCH,), jnp.int32),
        pltpu.VMEM((BCH,), jnp.float32),
        pltpu.SemaphoreType.DMA,
        pltpu.SemaphoreType.DMA,
        pltpu.SemaphoreType.DMA,
    ],
)


def _step_body(xs_hbm, an_hbm, a_hbm, sb_hbm, db_hbm, cb_hbm, out_hbm,
               acc_v, r0_v, r1_v, s0_v, s1_v, d0_v, d1_v,
               c0_v, c1_v, aa_v, sg0, sg1, st0, st1):
    sid = lax.axis_index("s")
    cid = lax.axis_index("c")
    wid = cid * NS + sid
    nbase = pl.multiple_of(wid * NTB, 64)
    rows = (r0_v, r1_v)
    sB = (s0_v, s1_v)
    dB = (d0_v, d1_v)
    cB = (c0_v, c1_v)
    sem_g = (sg0, sg1)
    sem_st = (st0, st1)

    def issue_stage(ch, q):
        off = wid * BCH + ch * ECH
        pltpu.async_copy(sb_hbm.at[pl.ds(off, ECH)], sB[q], sem_st[q])
        pltpu.async_copy(db_hbm.at[pl.ds(off, ECH)], dB[q], sem_st[q])
        pltpu.async_copy(cb_hbm.at[pl.ds(off, ECH)], cB[q], sem_st[q])

    def wait_stage(q):
        pltpu.make_async_copy(sb_hbm.at[pl.ds(0, ECH)], sB[q], sem_st[q]).wait()
        pltpu.make_async_copy(db_hbm.at[pl.ds(0, ECH)], dB[q], sem_st[q]).wait()
        pltpu.make_async_copy(cb_hbm.at[pl.ds(0, ECH)], cB[q], sem_st[q]).wait()

    def issue_gather(p, q):
        pltpu.async_copy(xs_hbm.at[sB[q]], rows[p], sem_g[p])

    def wait_gather(p):
        pltpu.make_async_copy(out_hbm.at[pl.ds(0, ECH)], rows[p],
                              sem_g[p]).wait()

    @pl.loop(0, NTB)
    def _zacc(r):
        for j in range(D // 16):
            acc_v[r, pl.ds(j * 16, 16)] = jnp.zeros((16,), jnp.float32)

    pltpu.sync_copy(a_hbm.at[pl.ds(nbase, NTB)], aa_v)

    issue_stage(0, 0)
    issue_stage(1, 1)
    wait_stage(0)
    issue_gather(0, 0)

    @pl.loop(0, NCH // 2)
    def _ring(t):
        for p in range(2):
            ch = t * 2 + p
            wait_gather(p)

            @pl.when(ch + 1 < NCH)
            def _pre_gather():
                wait_stage(1 - p)
                issue_gather(1 - p, 1 - p)

            @pl.loop(0, ECH // 16)
            def _scale(g):
                e16 = _iota16() + g * 16
                c16 = cB[p][pl.ds(g * 16, 16)]
                dl16 = dB[p][pl.ds(g * 16, 16)]
                for fb in range(0, D, 8):
                    cols = [jnp.full((16,), f, jnp.int32)
                            for f in range(fb, fb + 8)]
                    vs = [plsc.load_gather(rows[p], [e16, col])
                          for col in cols]
                    for col, v in zip(cols, vs):
                        plsc.addupdate_scatter(acc_v, [dl16, col], v * c16)

            @pl.when(ch + 2 < NCH)
            def _pre_stage():
                issue_stage(ch + 2, p)

    for r0, ln in ((0, ECH), (ECH, ECH), (2 * ECH, NTB - 2 * ECH)):
        pltpu.sync_copy(xs_hbm.at[pl.ds(nbase + r0, ln)],
                        r0_v.at[pl.ds(0, ln)])
        pltpu.sync_copy(an_hbm.at[pl.ds(nbase + r0, ln)],
                        r1_v.at[pl.ds(0, ln)])

        @pl.loop(0, ln // 16)
        def _upd(g):
            l16 = _iota16() + g * 16
            r16 = l16 + r0
            a16 = aa_v[pl.ds(r0 + g * 16, 16)]
            aa16 = a16 * a16
            for fb in range(0, D, 4):
                cols = [jnp.full((16,), f, jnp.int32)
                        for f in range(fb, fb + 4)]
                ags = [plsc.load_gather(acc_v, [r16, col]) for col in cols]
                xos = [plsc.load_gather(r0_v, [l16, col]) for col in cols]
                ans = [plsc.load_gather(r1_v, [l16, col]) for col in cols]
                for col, ag, xo, an in zip(cols, ags, xos, ans):
                    xn = 0.9 * (ag + aa16 * xo) + 0.1 * an
                    plsc.store_scatter(acc_v, [r16, col], xn)

    pltpu.sync_copy(acc_v, out_hbm.at[pl.ds(nbase, NTB)])


_step = pl.kernel(
    _step_body,
    out_type=jax.ShapeDtypeStruct((NP, D), jnp.float32),
    mesh=_mesh,
    compiler_params=_sc_params,
    scratch_types=[
        pltpu.VMEM((NTB, D), jnp.float32),
        pltpu.VMEM((ECH, D), jnp.float32),
        pltpu.VMEM((ECH, D), jnp.float32),
        pltpu.VMEM((ECH,), jnp.int32),
        pltpu.VMEM((ECH,), jnp.int32),
        pltpu.VMEM((ECH,), jnp.int32),
        pltpu.VMEM((ECH,), jnp.int32),
        pltpu.VMEM((ECH,), jnp.float32),
        pltpu.VMEM((ECH,), jnp.float32),
        pltpu.VMEM((NTB,), jnp.float32),
        pltpu.SemaphoreType.DMA,
        pltpu.SemaphoreType.DMA,
        pltpu.SemaphoreType.DMA,
        pltpu.SemaphoreType.DMA,
    ],
)


def kernel(x, edge_index, edge_attr, W1, b1, W2, b2):
    x_pad = jnp.pad(x, ((0, NP - N_NODES), (0, 0)))
    n_pad = E_PAD - N_EDGES
    pad_dst = (jnp.arange(n_pad, dtype=jnp.int32) % NW) * NTB
    src = jnp.pad(edge_index[0], (0, n_pad))
    dst = jnp.concatenate([edge_index[1], pad_dst])
    w = jnp.pad(edge_attr, (0, n_pad))

    h = _mlp(x_pad, W1, b1, W2, b2)
    a, srcB, dlB, cB = _prep(src, dst, w,
                             dst.reshape(-1, ECH), w.reshape(-1, ECH))

    xs = h
    for _layer in range(2):
        anchor = xs
        for _ in range(APPNP_K):
            xs = _step(xs, anchor, a, srcB, dlB, cB)

    return xs[:N_NODES]


# speedup vs baseline: 2.4971x; 2.4971x over previous
import functools

import jax
import jax.numpy as jnp
from jax import lax
from jax.experimental import pallas as pl
from jax.experimental.pallas import tpu as pltpu
from jax.experimental.pallas import tpu_sc as plsc

N_NODES = 10000
N_EDGES = 320000
INP_DIM = 128
HID_DIM = 256
OUT_DIM = 128
APPNP_K = 10
APPNP_ALPHA = 0.1

NC = 2
NS = 16
D = 128
NP = 10240
NPT = NP // NS
ECH = 128
E_PAD = 344064
EPT = E_PAD // NS
EPW = E_PAD // (NC * NS)
NCH = EPW // ECH
ECH_N = 64

_mesh = plsc.VectorSubcoreMesh(core_axis_name="c", subcore_axis_name="s")
_sc_params = pltpu.CompilerParams(needs_layout_passes=False)


def _iota16():
    return lax.iota(jnp.int32, 16)


def _rsqrt16(d):
    y = jnp.full((16,), 0.1, jnp.float32)
    for _ in range(12):
        y = y * (1.5 - 0.5 * d * y * y)
    return y


_MLP_BLK = 1024


def _mlp_body(x_ref, w1_ref, b1_ref, w2_ref, b2_ref, out_ref):
    xb = x_ref[...]
    h1 = lax.dot_general(xb, w1_ref[...], (((1,), (1,)), ((), ())),
                         preferred_element_type=jnp.float32) + b1_ref[...]
    h1 = jnp.where(h1 > 0, h1, jnp.exp(jnp.minimum(h1, 0.0)) - 1.0)
    out_ref[...] = lax.dot_general(h1, w2_ref[...], (((1,), (1,)), ((), ())),
                                   preferred_element_type=jnp.float32) + b2_ref[...]


def _mlp(x_pad, W1, b1, W2, b2):
    return pl.pallas_call(
        _mlp_body,
        grid=(NP // _MLP_BLK,),
        in_specs=[
            pl.BlockSpec((_MLP_BLK, INP_DIM), lambda i: (i, 0)),
            pl.BlockSpec((HID_DIM, INP_DIM), lambda i: (0, 0)),
            pl.BlockSpec((1, HID_DIM), lambda i: (0, 0)),
            pl.BlockSpec((OUT_DIM, HID_DIM), lambda i: (0, 0)),
            pl.BlockSpec((1, OUT_DIM), lambda i: (0, 0)),
        ],
        out_specs=pl.BlockSpec((_MLP_BLK, D), lambda i: (i, 0)),
        out_shape=jax.ShapeDtypeStruct((NP, D), jnp.float32),
    )(x_pad, W1, b1.reshape(1, HID_DIM), W2, b2.reshape(1, OUT_DIM))


def _prep_body(src_hbm, dst_hbm, w_hbm, a_hbm, c_hbm,
               deg_sp, a_sp, i1_v, f1_v, zb_v, av_v, i3_v, i4_v, f3_v, f4_v):
    sid = lax.axis_index("s")
    cid = lax.axis_index("c")
    base_n = sid * NPT

    for g in range(NPT // 16):
        zb_v[pl.ds(g * 16, 16)] = jnp.zeros((16,), jnp.float32)
    pltpu.sync_copy(zb_v, deg_sp.at[pl.ds(base_n, NPT)])
    plsc.subcore_barrier()

    @pl.loop(0, EPT // ECH)
    def _deg_chunk(k):
        off = sid * EPT + k * ECH
        pltpu.sync_copy(dst_hbm.at[pl.ds(off, ECH)], i1_v)
        pltpu.sync_copy(w_hbm.at[pl.ds(off, ECH)], f1_v)
        pltpu.sync_copy(f1_v, deg_sp.at[i1_v], add=True)

    plsc.subcore_barrier()

    pltpu.sync_copy(deg_sp.at[pl.ds(base_n, NPT)], zb_v)
    for g in range(NPT // 16):
        d16 = zb_v[pl.ds(g * 16, 16)] + 1.0
        zb_v[pl.ds(g * 16, 16)] = _rsqrt16(d16)
    pltpu.sync_copy(zb_v, a_sp.at[pl.ds(base_n, NPT)])
    pltpu.sync_copy(zb_v, a_hbm.at[pl.ds(base_n, NPT)])
    plsc.subcore_barrier()

    pltpu.sync_copy(a_sp, av_v)

    @pl.loop(0, EPW // ECH_N)
    def _norm_chunk(k):
        off = (cid * NS + sid) * EPW + k * ECH_N
        pltpu.sync_copy(src_hbm.at[pl.ds(off, ECH_N)], i3_v)
        pltpu.sync_copy(dst_hbm.at[pl.ds(off, ECH_N)], i4_v)
        pltpu.sync_copy(w_hbm.at[pl.ds(off, ECH_N)], f3_v)
        for g in range(ECH_N // 16):
            s16 = i3_v[pl.ds(g * 16, 16)]
            d16 = i4_v[pl.ds(g * 16, 16)]
            asrc = plsc.load_gather(av_v, [s16])
            adst = plsc.load_gather(av_v, [d16])
            f4_v[pl.ds(g * 16, 16)] = asrc * adst * f3_v[pl.ds(g * 16, 16)]
        pltpu.sync_copy(f4_v, c_hbm.at[pl.ds(off, ECH_N)])


_prep = pl.kernel(
    _prep_body,
    out_type=(
        jax.ShapeDtypeStruct((NP,), jnp.float32),
        jax.ShapeDtypeStruct((E_PAD,), jnp.float32),
    ),
    mesh=_mesh,
    compiler_params=_sc_params,
    scratch_types=[
        pltpu.VMEM_SHARED((NP,), jnp.float32),
        pltpu.VMEM_SHARED((NP,), jnp.float32),
        pltpu.VMEM((ECH,), jnp.int32),
        pltpu.VMEM((ECH,), jnp.float32),
        pltpu.VMEM((NPT,), jnp.float32),
        pltpu.VMEM((NP,), jnp.float32),
        pltpu.VMEM((ECH_N,), jnp.int32),
        pltpu.VMEM((ECH_N,), jnp.int32),
        pltpu.VMEM((ECH_N,), jnp.float32),
        pltpu.VMEM((ECH_N,), jnp.float32),
    ],
)


def _edge_body(xs_hbm, sd_hbm, c_hbm, p_hbm,
               acc_sp, r0_v, r1_v, s0_v, s1_v, s2_v, c0_v, c1_v, c2_v,
               sg0, sg1, st0, st1, st2, sc0, sc1):
    sid = lax.axis_index("s")
    cid = lax.axis_index("c")
    base_n = sid * NPT
    wid = cid * NS + sid
    rows = (r0_v, r1_v)
    sdv = (s0_v, s1_v, s2_v)
    cv = (c0_v, c1_v, c2_v)
    sem_g = (sg0, sg1)
    sem_st = (st0, st1, st2)
    sem_sc = (sc0, sc1)

    def issue_stage(ch, q):
        pltpu.async_copy(sd_hbm.at[pl.ds((wid * NCH + ch) * 2, 2)],
                         sdv[q], sem_st[q])
        pltpu.async_copy(c_hbm.at[pl.ds(wid * EPW + ch * ECH, ECH)],
                         cv[q], sem_st[q])

    def wait_stage(q):
        pltpu.make_async_copy(sd_hbm.at[pl.ds(0, 2)], sdv[q], sem_st[q]).wait()
        pltpu.make_async_copy(c_hbm.at[pl.ds(0, ECH)], cv[q], sem_st[q]).wait()

    def issue_gather(p, q):
        pltpu.async_copy(xs_hbm.at[sdv[q].at[0]], rows[p], sem_g[p])

    def wait_gather(p):
        pltpu.make_async_copy(p_hbm.at[pl.ds(0, ECH)], rows[p], sem_g[p]).wait()

    def issue_scatter(p, q):
        pltpu.async_copy(rows[p], acc_sp.at[sdv[q].at[1]], sem_sc[p], add=True)

    def wait_scatter(p):
        pltpu.make_async_copy(rows[p], acc_sp.at[pl.ds(0, ECH)],
                              sem_sc[p]).wait()

    @pl.loop(0, ECH)
    def _zrow(r):
        for j in range(D // 16):
            r0_v[r, pl.ds(j * 16, 16)] = jnp.zeros((16,), jnp.float32)

    for part in range(NPT // ECH):
        pltpu.sync_copy(r0_v, acc_sp.at[pl.ds(base_n + part * ECH, ECH)])

    issue_stage(0, 0)
    issue_stage(1, 1)
    wait_stage(0)
    issue_gather(0, 0)
    plsc.subcore_barrier()

    @pl.loop(0, NCH // 6)
    def _ring(t):
        for k in range(6):
            ch = t * 6 + k
            p, q = k % 2, k % 3
            wait_gather(p)

            @pl.when(ch >= 1)
            def _drain_prev_scatter():
                wait_scatter(1 - p)

            @pl.when(ch + 1 < NCH)
            def _pre_gather():
                wait_stage((q + 1) % 3)
                issue_gather(1 - p, (q + 1) % 3)

            @pl.when(ch + 2 < NCH)
            def _pre_stage():
                issue_stage(ch + 2, (q + 2) % 3)

            @pl.loop(0, ECH // 16)
            def _scale(g):
                e16 = _iota16() + g * 16
                c16 = cv[q][pl.ds(g * 16, 16)]
                for fb in range(0, D, 8):
                    cols = [jnp.full((16,), f, jnp.int32)
                            for f in range(fb, fb + 8)]
                    vs = [plsc.load_gather(rows[p], [e16, col])
                          for col in cols]
                    for col, v in zip(cols, vs):
                        plsc.store_scatter(rows[p], [e16, col], v * c16)

            issue_scatter(p, q)

    wait_scatter((NCH - 1) % 2)
    plsc.subcore_barrier()
    pltpu.sync_copy(acc_sp.at[pl.ds(base_n, NPT)],
                    p_hbm.at[pl.ds(cid * NP + base_n, NPT)])


_edge = pl.kernel(
    _edge_body,
    out_type=jax.ShapeDtypeStruct((NC * NP, D), jnp.float32),
    mesh=_mesh,
    compiler_params=_sc_params,
    scratch_types=[
        pltpu.VMEM_SHARED((NP, D), jnp.float32),
        pltpu.VMEM((ECH, D), jnp.float32),
        pltpu.VMEM((ECH, D), jnp.float32),
        pltpu.VMEM((2, ECH), jnp.int32),
        pltpu.VMEM((2, ECH), jnp.int32),
        pltpu.VMEM((2, ECH), jnp.int32),
        pltpu.VMEM((ECH,), jnp.float32),
        pltpu.VMEM((ECH,), jnp.float32),
        pltpu.VMEM((ECH,), jnp.float32),
        pltpu.SemaphoreType.DMA,
        pltpu.SemaphoreType.DMA,
        pltpu.SemaphoreType.DMA,
        pltpu.SemaphoreType.DMA,
        pltpu.SemaphoreType.DMA,
        pltpu.SemaphoreType.DMA,
        pltpu.SemaphoreType.DMA,
    ],
)


_UPD_BLK = 1024


def _update_body(p_ref, x_ref, h_ref, a_ref, out_ref):
    aa = a_ref[...] * a_ref[...]
    agg = p_ref[0] + p_ref[1] + aa * x_ref[...]
    out_ref[...] = (1.0 - APPNP_ALPHA) * agg + APPNP_ALPHA * h_ref[...]


def _update(p, x, h, a_col):
    return pl.pallas_call(
        _update_body,
        grid=(NP // _UPD_BLK,),
        in_specs=[
            pl.BlockSpec((NC, _UPD_BLK, D), lambda i: (0, i, 0)),
            pl.BlockSpec((_UPD_BLK, D), lambda i: (i, 0)),
            pl.BlockSpec((_UPD_BLK, D), lambda i: (i, 0)),
            pl.BlockSpec((_UPD_BLK, 1), lambda i: (i, 0)),
        ],
        out_specs=pl.BlockSpec((_UPD_BLK, D), lambda i: (i, 0)),
        out_shape=jax.ShapeDtypeStruct((NP, D), jnp.float32),
    )(p, x, h, a_col)


def kernel(x, edge_index, edge_attr, W1, b1, W2, b2):
    x_pad = jnp.pad(x, ((0, NP - N_NODES), (0, 0)))
    src = jnp.pad(edge_index[0], (0, E_PAD - N_EDGES))
    dst = jnp.pad(edge_index[1], (0, E_PAD - N_EDGES))
    w = jnp.pad(edge_attr, (0, E_PAD - N_EDGES))

    h = _mlp(x_pad, W1, b1, W2, b2)
    a, c = _prep(src, dst, w)
    a_col = a.reshape(NP, 1)
    sd = jnp.concatenate(
        [src.reshape(-1, 1, ECH), dst.reshape(-1, 1, ECH)], axis=1
    ).reshape(-1, ECH)

    xs = h
    for _layer in range(2):
        anchor = xs
        for _ in range(APPNP_K):
            p = _edge(xs, sd, c)
            xs = _update(p.reshape(NC, NP, D), xs, anchor, a_col)

    return xs[:N_NODES]


# speedup vs baseline: 2.6419x; 1.0580x over previous
import functools

import jax
import jax.numpy as jnp
from jax import lax
from jax.experimental import pallas as pl
from jax.experimental.pallas import tpu as pltpu
from jax.experimental.pallas import tpu_sc as plsc

N_NODES = 10000
N_EDGES = 320000
INP_DIM = 128
HID_DIM = 256
OUT_DIM = 128
APPNP_K = 10
APPNP_ALPHA = 0.1

NC = 2
NS = 16
D = 128
NP = 10240
NPT = NP // NS
ECH = 128
E_PAD = 344064
EPT = E_PAD // NS
EPW = E_PAD // (NC * NS)
NCH = EPW // ECH
ECH_N = 64

_mesh = plsc.VectorSubcoreMesh(core_axis_name="c", subcore_axis_name="s")
_sc_params = pltpu.CompilerParams(needs_layout_passes=False)


def _iota16():
    return lax.iota(jnp.int32, 16)


def _rsqrt16(d):
    y = jnp.full((16,), 0.1, jnp.float32)
    for _ in range(12):
        y = y * (1.5 - 0.5 * d * y * y)
    return y


_MLP_BLK = 1024


def _mlp_body(x_ref, w1_ref, b1_ref, w2_ref, b2_ref, out_ref):
    xb = x_ref[...]
    h1 = lax.dot_general(xb, w1_ref[...], (((1,), (1,)), ((), ())),
                         preferred_element_type=jnp.float32) + b1_ref[...]
    h1 = jnp.where(h1 > 0, h1, jnp.exp(jnp.minimum(h1, 0.0)) - 1.0)
    out_ref[...] = lax.dot_general(h1, w2_ref[...], (((1,), (1,)), ((), ())),
                                   preferred_element_type=jnp.float32) + b2_ref[...]


def _mlp(x_pad, W1, b1, W2, b2):
    return pl.pallas_call(
        _mlp_body,
        grid=(NP // _MLP_BLK,),
        in_specs=[
            pl.BlockSpec((_MLP_BLK, INP_DIM), lambda i: (i, 0)),
            pl.BlockSpec((HID_DIM, INP_DIM), lambda i: (0, 0)),
            pl.BlockSpec((1, HID_DIM), lambda i: (0, 0)),
            pl.BlockSpec((OUT_DIM, HID_DIM), lambda i: (0, 0)),
            pl.BlockSpec((1, OUT_DIM), lambda i: (0, 0)),
        ],
        out_specs=pl.BlockSpec((_MLP_BLK, D), lambda i: (i, 0)),
        out_shape=jax.ShapeDtypeStruct((NP, D), jnp.float32),
    )(x_pad, W1, b1.reshape(1, HID_DIM), W2, b2.reshape(1, OUT_DIM))


def _prep_body(src_hbm, dst_hbm, w_hbm, a_hbm, c_hbm,
               deg_sp, a_sp, i1_v, f1_v, zb_v, av_v, i3_v, i4_v, f3_v, f4_v):
    sid = lax.axis_index("s")
    cid = lax.axis_index("c")
    base_n = sid * NPT

    for g in range(NPT // 16):
        zb_v[pl.ds(g * 16, 16)] = jnp.zeros((16,), jnp.float32)
    pltpu.sync_copy(zb_v, deg_sp.at[pl.ds(base_n, NPT)])
    plsc.subcore_barrier()

    @pl.loop(0, EPT // ECH)
    def _deg_chunk(k):
        off = sid * EPT + k * ECH
        pltpu.sync_copy(dst_hbm.at[pl.ds(off, ECH)], i1_v)
        pltpu.sync_copy(w_hbm.at[pl.ds(off, ECH)], f1_v)
        pltpu.sync_copy(f1_v, deg_sp.at[i1_v], add=True)

    plsc.subcore_barrier()

    pltpu.sync_copy(deg_sp.at[pl.ds(base_n, NPT)], zb_v)
    for g in range(NPT // 16):
        d16 = zb_v[pl.ds(g * 16, 16)] + 1.0
        zb_v[pl.ds(g * 16, 16)] = _rsqrt16(d16)
    pltpu.sync_copy(zb_v, a_sp.at[pl.ds(base_n, NPT)])
    pltpu.sync_copy(zb_v, a_hbm.at[pl.ds(base_n, NPT)])
    plsc.subcore_barrier()

    pltpu.sync_copy(a_sp, av_v)

    @pl.loop(0, EPW // ECH_N)
    def _norm_chunk(k):
        off = (cid * NS + sid) * EPW + k * ECH_N
        pltpu.sync_copy(src_hbm.at[pl.ds(off, ECH_N)], i3_v)
        pltpu.sync_copy(dst_hbm.at[pl.ds(off, ECH_N)], i4_v)
        pltpu.sync_copy(w_hbm.at[pl.ds(off, ECH_N)], f3_v)
        for g in range(ECH_N // 16):
            s16 = i3_v[pl.ds(g * 16, 16)]
            d16 = i4_v[pl.ds(g * 16, 16)]
            asrc = plsc.load_gather(av_v, [s16])
            adst = plsc.load_gather(av_v, [d16])
            f4_v[pl.ds(g * 16, 16)] = asrc * adst * f3_v[pl.ds(g * 16, 16)]
        pltpu.sync_copy(f4_v, c_hbm.at[pl.ds(off, ECH_N)])


_prep = pl.kernel(
    _prep_body,
    out_type=(
        jax.ShapeDtypeStruct((NP,), jnp.float32),
        jax.ShapeDtypeStruct((E_PAD,), jnp.float32),
    ),
    mesh=_mesh,
    compiler_params=_sc_params,
    scratch_types=[
        pltpu.VMEM_SHARED((NP,), jnp.float32),
        pltpu.VMEM_SHARED((NP,), jnp.float32),
        pltpu.VMEM((ECH,), jnp.int32),
        pltpu.VMEM((ECH,), jnp.float32),
        pltpu.VMEM((NPT,), jnp.float32),
        pltpu.VMEM((NP,), jnp.float32),
        pltpu.VMEM((ECH_N,), jnp.int32),
        pltpu.VMEM((ECH_N,), jnp.int32),
        pltpu.VMEM((ECH_N,), jnp.float32),
        pltpu.VMEM((ECH_N,), jnp.float32),
    ],
)


def _edge_body(xs_hbm, sd_hbm, c_hbm, p_hbm,
               acc_sp, r0_v, r1_v, s0_v, s1_v, s2_v, c0_v, c1_v, c2_v,
               sg0, sg1, st0, st1, st2, sc0, sc1):
    sid = lax.axis_index("s")
    cid = lax.axis_index("c")
    base_n = sid * NPT
    wid = cid * NS + sid
    rows = (r0_v, r1_v)
    sdv = (s0_v, s1_v, s2_v)
    cv = (c0_v, c1_v, c2_v)
    sem_g = (sg0, sg1)
    sem_st = (st0, st1, st2)
    sem_sc = (sc0, sc1)

    def issue_stage(ch, q):
        pltpu.async_copy(sd_hbm.at[pl.ds((wid * NCH + ch) * 2, 2)],
                         sdv[q], sem_st[q])
        pltpu.async_copy(c_hbm.at[pl.ds(wid * EPW + ch * ECH, ECH)],
                         cv[q], sem_st[q])

    def wait_stage(q):
        pltpu.make_async_copy(sd_hbm.at[pl.ds(0, 2)], sdv[q], sem_st[q]).wait()
        pltpu.make_async_copy(c_hbm.at[pl.ds(0, ECH)], cv[q], sem_st[q]).wait()

    def issue_gather(p, q):
        pltpu.async_copy(xs_hbm.at[sdv[q].at[0]], rows[p], sem_g[p])

    def wait_gather(p):
        pltpu.make_async_copy(p_hbm.at[pl.ds(0, ECH)], rows[p], sem_g[p]).wait()

    def issue_scatter(p, q):
        pltpu.async_copy(rows[p], acc_sp.at[sdv[q].at[1]], sem_sc[p], add=True)

    def wait_scatter(p):
        pltpu.make_async_copy(rows[p], acc_sp.at[pl.ds(0, ECH)],
                              sem_sc[p]).wait()

    @pl.loop(0, ECH)
    def _zrow(r):
        for j in range(D // 16):
            r0_v[r, pl.ds(j * 16, 16)] = jnp.zeros((16,), jnp.float32)

    for part in range(NPT // ECH):
        pltpu.sync_copy(r0_v, acc_sp.at[pl.ds(base_n + part * ECH, ECH)])

    issue_stage(0, 0)
    issue_stage(1, 1)
    wait_stage(0)
    issue_gather(0, 0)
    plsc.subcore_barrier()

    @pl.loop(0, NCH // 6)
    def _ring(t):
        for k in range(6):
            ch = t * 6 + k
            p, q = k % 2, k % 3
            wait_gather(p)

            @pl.when(ch >= 1)
            def _drain_prev_scatter():
                wait_scatter(1 - p)

            @pl.when(ch + 1 < NCH)
            def _pre_gather():
                wait_stage((q + 1) % 3)
                issue_gather(1 - p, (q + 1) % 3)

            @pl.when(ch + 2 < NCH)
            def _pre_stage():
                issue_stage(ch + 2, (q + 2) % 3)

            @pl.loop(0, ECH // 16)
            def _scale(g):
                e16 = _iota16() + g * 16
                c16 = cv[q][pl.ds(g * 16, 16)]
                for fb in range(0, D, 8):
                    cols = [(_iota16() + f) & (D - 1)
                            for f in range(fb, fb + 8)]
                    vs = [plsc.load_gather(rows[p], [e16, col])
                          for col in cols]
                    for col, v in zip(cols, vs):
                        plsc.store_scatter(rows[p], [e16, col], v * c16)

            issue_scatter(p, q)

    wait_scatter((NCH - 1) % 2)
    plsc.subcore_barrier()
    pltpu.sync_copy(acc_sp.at[pl.ds(base_n, NPT)],
                    p_hbm.at[pl.ds(cid * NP + base_n, NPT)])


_edge = pl.kernel(
    _edge_body,
    out_type=jax.ShapeDtypeStruct((NC * NP, D), jnp.float32),
    mesh=_mesh,
    compiler_params=_sc_params,
    scratch_types=[
        pltpu.VMEM_SHARED((NP, D), jnp.float32),
        pltpu.VMEM((ECH, D), jnp.float32),
        pltpu.VMEM((ECH, D), jnp.float32),
        pltpu.VMEM((2, ECH), jnp.int32),
        pltpu.VMEM((2, ECH), jnp.int32),
        pltpu.VMEM((2, ECH), jnp.int32),
        pltpu.VMEM((ECH,), jnp.float32),
        pltpu.VMEM((ECH,), jnp.float32),
        pltpu.VMEM((ECH,), jnp.float32),
        pltpu.SemaphoreType.DMA,
        pltpu.SemaphoreType.DMA,
        pltpu.SemaphoreType.DMA,
        pltpu.SemaphoreType.DMA,
        pltpu.SemaphoreType.DMA,
        pltpu.SemaphoreType.DMA,
        pltpu.SemaphoreType.DMA,
    ],
)


_UPD_BLK = 1024


def _update_body(p_ref, x_ref, h_ref, a_ref, out_ref):
    aa = a_ref[...] * a_ref[...]
    agg = p_ref[0] + p_ref[1] + aa * x_ref[...]
    out_ref[...] = (1.0 - APPNP_ALPHA) * agg + APPNP_ALPHA * h_ref[...]


def _update(p, x, h, a_col):
    return pl.pallas_call(
        _update_body,
        grid=(NP // _UPD_BLK,),
        in_specs=[
            pl.BlockSpec((NC, _UPD_BLK, D), lambda i: (0, i, 0)),
            pl.BlockSpec((_UPD_BLK, D), lambda i: (i, 0)),
            pl.BlockSpec((_UPD_BLK, D), lambda i: (i, 0)),
            pl.BlockSpec((_UPD_BLK, 1), lambda i: (i, 0)),
        ],
        out_specs=pl.BlockSpec((_UPD_BLK, D), lambda i: (i, 0)),
        out_shape=jax.ShapeDtypeStruct((NP, D), jnp.float32),
    )(p, x, h, a_col)


def kernel(x, edge_index, edge_attr, W1, b1, W2, b2):
    x_pad = jnp.pad(x, ((0, NP - N_NODES), (0, 0)))
    src = jnp.pad(edge_index[0], (0, E_PAD - N_EDGES))
    dst = jnp.pad(edge_index[1], (0, E_PAD - N_EDGES))
    w = jnp.pad(edge_attr, (0, E_PAD - N_EDGES))

    h = _mlp(x_pad, W1, b1, W2, b2)
    a, c = _prep(src, dst, w)
    a_col = a.reshape(NP, 1)
    sd = jnp.concatenate(
        [src.reshape(-1, 1, ECH), dst.reshape(-1, 1, ECH)], axis=1
    ).reshape(-1, ECH)

    xs = h
    for _layer in range(2):
        anchor = xs
        for _ in range(APPNP_K):
            p = _edge(xs, sd, c)
            xs = _update(p.reshape(NC, NP, D), xs, anchor, a_col)

    return xs[:N_NODES]
